# Initial kernel scaffold; baseline (speedup 1.0000x reference)
import jax
import jax.numpy as jnp
from jax.experimental import pallas as pl


def kernel(state, U):
    raise NotImplementedError("write your pallas kernel here")


# speedup vs baseline: 27.2118x; 27.2118x over previous
import functools

import jax
import jax.numpy as jnp
from jax import lax
from jax.experimental import pallas as pl
from jax.experimental.pallas import tpu as pltpu
from jax.experimental.pallas import tpu_sc as plsc

_NQ = 23
_DIM = 2 ** _NQ
_Q = _DIM // 4
_HALF = _DIM // 2
_NC = 2
_NS = 16
_NW = _NC * _NS
_BLEND_W = _Q // _NW
_COPY_W = _HALF // _NW
_CH = 8192
_NCHUNK = _BLEND_W // _CH
_LANES = 16

_mesh = plsc.VectorSubcoreMesh(core_axis_name="c", subcore_axis_name="s")


@functools.partial(
    pl.kernel,
    mesh=_mesh,
    out_type=jax.ShapeDtypeStruct((_DIM,), jnp.float32),
    scratch_types=[
        pltpu.VMEM((4, _LANES), jnp.float32),
        pltpu.VMEM((_CH,), jnp.float32),
        pltpu.VMEM((_CH,), jnp.float32),
    ],
)
def _cgate(state_hbm, u_hbm, out_hbm, u_v, a0_v, a1_v):
    wid = lax.axis_index("s") * _NC + lax.axis_index("c")

    coff = wid * _COPY_W
    pltpu.sync_copy(state_hbm.at[pl.ds(coff, _COPY_W)],
                    out_hbm.at[pl.ds(coff, _COPY_W)])

    pltpu.sync_copy(u_hbm, u_v)
    u00 = u_v[0]
    u01 = u_v[1]
    u10 = u_v[2]
    u11 = u_v[3]

    base = wid * _BLEND_W
    for k in range(_NCHUNK):
        off0 = 2 * _Q + base + k * _CH
        off1 = 3 * _Q + base + k * _CH
        pltpu.sync_copy(state_hbm.at[pl.ds(off0, _CH)], a0_v)
        pltpu.sync_copy(state_hbm.at[pl.ds(off1, _CH)], a1_v)

        def body(i, carry):
            sl = pl.ds(i * _LANES, _LANES)
            a0 = a0_v[sl]
            a1 = a1_v[sl]
            a0_v[sl] = u00 * a0 + u01 * a1
            a1_v[sl] = u10 * a0 + u11 * a1
            return carry

        lax.fori_loop(0, _CH // _LANES, body, 0)

        pltpu.sync_copy(a0_v, out_hbm.at[pl.ds(off0, _CH)])
        pltpu.sync_copy(a1_v, out_hbm.at[pl.ds(off1, _CH)])


def kernel(state, U):
    u_rows = jnp.tile(U.astype(jnp.float32).reshape(4, 1), (1, _LANES))
    return _cgate(state, u_rows)


# speedup vs baseline: 29.6560x; 1.0898x over previous
import functools

import jax
import jax.numpy as jnp
from jax import lax
from jax.experimental import pallas as pl
from jax.experimental.pallas import tpu as pltpu
from jax.experimental.pallas import tpu_sc as plsc

_NQ = 23
_DIM = 2 ** _NQ
_Q = _DIM // 4
_HALF = _DIM // 2
_NC = 2
_NS = 16
_NW = _NC * _NS
_BLEND_W = _Q // _NW
_COPY_W = _HALF // _NW
_CH = 8192
_NCHUNK = _BLEND_W // _CH
_LANES = 16
_UNROLL = 4

_mesh = plsc.VectorSubcoreMesh(core_axis_name="c", subcore_axis_name="s")


@functools.partial(
    pl.kernel,
    mesh=_mesh,
    out_type=jax.ShapeDtypeStruct((_DIM,), jnp.float32),
    scratch_types=[
        pltpu.VMEM((4, _LANES), jnp.float32),
        pltpu.VMEM((2, _CH), jnp.float32),
        pltpu.VMEM((2, _CH), jnp.float32),
        pltpu.VMEM((2, _CH), jnp.float32),
        pltpu.VMEM((2, _CH), jnp.float32),
        pltpu.SemaphoreType.DMA,
        pltpu.SemaphoreType.DMA,
        pltpu.SemaphoreType.DMA,
        pltpu.SemaphoreType.DMA,
        pltpu.SemaphoreType.DMA,
    ],
)
def _cgate(state_hbm, u_hbm, out_hbm, u_v, a0_v, a1_v, o0_v, o1_v,
           si0, si1, so0, so1, scp):
    si = (si0, si1)
    so = (so0, so1)
    wid = lax.axis_index("s") * _NC + lax.axis_index("c")

    coff = wid * _COPY_W
    cp = pltpu.async_copy(state_hbm.at[pl.ds(coff, _COPY_W)],
                          out_hbm.at[pl.ds(coff, _COPY_W)], scp)

    pltpu.sync_copy(u_hbm, u_v)
    u00 = u_v[0]
    u01 = u_v[1]
    u10 = u_v[2]
    u11 = u_v[3]

    base = wid * _BLEND_W

    def off0(k):
        return 2 * _Q + base + k * _CH

    def off1(k):
        return 3 * _Q + base + k * _CH

    h_in = {}
    h_out = {}

    def start_in(k):
        b = k & 1
        h_in[k] = (
            pltpu.async_copy(state_hbm.at[pl.ds(off0(k), _CH)],
                             a0_v.at[b], si[b]),
            pltpu.async_copy(state_hbm.at[pl.ds(off1(k), _CH)],
                             a1_v.at[b], si[b]),
        )

    start_in(0)
    start_in(1)

    for k in range(_NCHUNK):
        b = k & 1
        if k >= 2:
            for h in h_out[k - 2]:
                h.wait()
        for h in h_in[k]:
            h.wait()

        a0b = a0_v.at[b]
        a1b = a1_v.at[b]
        o0b = o0_v.at[b]
        o1b = o1_v.at[b]

        def body(i, carry):
            for j in range(_UNROLL):
                sl = pl.ds(i * (_LANES * _UNROLL) + j * _LANES, _LANES)
                a0 = a0b[sl]
                a1 = a1b[sl]
                o0b[sl] = u00 * a0 + u01 * a1
                o1b[sl] = u10 * a0 + u11 * a1
            return carry

        lax.fori_loop(0, _CH // (_LANES * _UNROLL), body, 0)

        h_out[k] = (
            pltpu.async_copy(o0b, out_hbm.at[pl.ds(off0(k), _CH)], so[b]),
            pltpu.async_copy(o1b, out_hbm.at[pl.ds(off1(k), _CH)], so[b]),
        )
        if k + 2 < _NCHUNK:
            start_in(k + 2)

    for k in (_NCHUNK - 2, _NCHUNK - 1):
        for h in h_out[k]:
            h.wait()
    cp.wait()


def kernel(state, U):
    u_rows = jnp.tile(U.astype(jnp.float32).reshape(4, 1), (1, _LANES))
    return _cgate(state, u_rows)


# speedup vs baseline: 315.1399x; 10.6265x over previous
import functools

import jax
import jax.numpy as jnp
from jax import lax
from jax.experimental import pallas as pl
from jax.experimental.pallas import tpu as pltpu
from jax.experimental.pallas import tpu_sc as plsc

_NQ = 23
_DIM = 2 ** _NQ
_Q = _DIM // 4
_HALF = _DIM // 2
_NC = 2
_NS = 16
_NW = _NC * _NS
_BLEND_W = _Q // _NW
_COPY_W = _HALF // _NW
_CH = 8192
_NCHUNK = _BLEND_W // _CH
_CCH = 16384
_NCCH = _COPY_W // _CCH
_LANES = 16
_UNROLL = 4

_mesh = plsc.VectorSubcoreMesh(core_axis_name="c", subcore_axis_name="s")


@functools.partial(
    pl.kernel,
    mesh=_mesh,
    out_type=jax.ShapeDtypeStruct((_DIM,), jnp.float32),
    scratch_types=[
        pltpu.VMEM((4, _LANES), jnp.float32),
        pltpu.VMEM((2, _CH), jnp.float32),
        pltpu.VMEM((2, _CH), jnp.float32),
        pltpu.VMEM((2, _CH), jnp.float32),
        pltpu.VMEM((2, _CH), jnp.float32),
        pltpu.VMEM((2, _CCH), jnp.float32),
        pltpu.SemaphoreType.DMA,
        pltpu.SemaphoreType.DMA,
        pltpu.SemaphoreType.DMA,
        pltpu.SemaphoreType.DMA,
        pltpu.SemaphoreType.DMA,
        pltpu.SemaphoreType.DMA,
        pltpu.SemaphoreType.DMA,
        pltpu.SemaphoreType.DMA,
    ],
)
def _cgate(state_hbm, u_hbm, out_hbm, u_v, a0_v, a1_v, o0_v, o1_v, c_v,
           si0, si1, so0, so1, ci0, ci1, co0, co1):
    si = (si0, si1)
    so = (so0, so1)
    ci = (ci0, ci1)
    co = (co0, co1)
    wid = lax.axis_index("s") * _NC + lax.axis_index("c")

    pltpu.sync_copy(u_hbm, u_v)
    u00 = u_v[0]
    u01 = u_v[1]
    u10 = u_v[2]
    u11 = u_v[3]

    base = wid * _BLEND_W
    coff = wid * _COPY_W

    def off0(k):
        return 2 * _Q + base + k * _CH

    def off1(k):
        return 3 * _Q + base + k * _CH

    def offc(k):
        return coff + k * _CCH

    h_in = {}
    h_out = {}
    h_cin = {}
    h_cout = {}

    def start_in(k):
        b = k & 1
        h_in[k] = (
            pltpu.async_copy(state_hbm.at[pl.ds(off0(k), _CH)],
                             a0_v.at[b], si[b]),
            pltpu.async_copy(state_hbm.at[pl.ds(off1(k), _CH)],
                             a1_v.at[b], si[b]),
        )

    def start_cin(k):
        b = k & 1
        h_cin[k] = pltpu.async_copy(state_hbm.at[pl.ds(offc(k), _CCH)],
                                    c_v.at[b], ci[b])

    start_cin(0)
    start_cin(1)
    start_in(0)
    start_in(1)

    for k in range(_NCHUNK):
        b = k & 1
        if k >= 2:
            for h in h_out[k - 2]:
                h.wait()
        for h in h_in[k]:
            h.wait()

        a0b = a0_v.at[b]
        a1b = a1_v.at[b]
        o0b = o0_v.at[b]
        o1b = o1_v.at[b]

        def body(i, carry):
            for j in range(_UNROLL):
                sl = pl.ds(i * (_LANES * _UNROLL) + j * _LANES, _LANES)
                a0 = a0b[sl]
                a1 = a1b[sl]
                o0b[sl] = u00 * a0 + u01 * a1
                o1b[sl] = u10 * a0 + u11 * a1
            return carry

        lax.fori_loop(0, _CH // (_LANES * _UNROLL), body, 0)

        h_out[k] = (
            pltpu.async_copy(o0b, out_hbm.at[pl.ds(off0(k), _CH)], so[b]),
            pltpu.async_copy(o1b, out_hbm.at[pl.ds(off1(k), _CH)], so[b]),
        )
        if k + 2 < _NCHUNK:
            start_in(k + 2)

        if k < _NCCH:
            if k >= 2:
                h_cout[k - 2].wait()
            h_cin[k].wait()
            h_cout[k] = pltpu.async_copy(c_v.at[b],
                                         out_hbm.at[pl.ds(offc(k), _CCH)],
                                         co[b])
            if k + 2 < _NCCH:
                start_cin(k + 2)

    for k in (_NCHUNK - 2, _NCHUNK - 1):
        for h in h_out[k]:
            h.wait()
    for k in (_NCCH - 2, _NCCH - 1):
        h_cout[k].wait()


def kernel(state, U):
    u_rows = jnp.tile(U.astype(jnp.float32).reshape(4, 1), (1, _LANES))
    return _cgate(state, u_rows)


# speedup vs baseline: 323.1673x; 1.0255x over previous
import functools

import jax
import jax.numpy as jnp
from jax import lax
from jax.experimental import pallas as pl
from jax.experimental.pallas import tpu as pltpu
from jax.experimental.pallas import tpu_sc as plsc

_NQ = 23
_DIM = 2 ** _NQ
_Q = _DIM // 4
_HALF = _DIM // 2
_NC = 2
_NS = 16
_NW = _NC * _NS
_BLEND_W = _Q // _NW
_COPY_W = _HALF // _NW
_CH = 8192
_NCHUNK = _BLEND_W // _CH
_CCH = 16384
_NCCH = _COPY_W // _CCH
_LANES = 16
_UNROLL = 8

_mesh = plsc.VectorSubcoreMesh(core_axis_name="c", subcore_axis_name="s")


@functools.partial(
    pl.kernel,
    mesh=_mesh,
    out_type=jax.ShapeDtypeStruct((_DIM,), jnp.float32),
    scratch_types=[
        pltpu.VMEM((4, _LANES), jnp.float32),
        pltpu.VMEM((2, _CH), jnp.float32),
        pltpu.VMEM((2, _CH), jnp.float32),
        pltpu.VMEM((2, _CH), jnp.float32),
        pltpu.VMEM((2, _CH), jnp.float32),
        pltpu.VMEM((2, _CCH), jnp.float32),
        pltpu.SemaphoreType.DMA,
        pltpu.SemaphoreType.DMA,
        pltpu.SemaphoreType.DMA,
        pltpu.SemaphoreType.DMA,
        pltpu.SemaphoreType.DMA,
        pltpu.SemaphoreType.DMA,
        pltpu.SemaphoreType.DMA,
        pltpu.SemaphoreType.DMA,
    ],
)
def _cgate(state_hbm, u_hbm, out_hbm, u_v, a0_v, a1_v, o0_v, o1_v, c_v,
           si0, si1, so0, so1, ci0, ci1, co0, co1):
    si = (si0, si1)
    so = (so0, so1)
    ci = (ci0, ci1)
    co = (co0, co1)
    wid = lax.axis_index("s") * _NC + lax.axis_index("c")

    pltpu.sync_copy(u_hbm, u_v)
    u00 = u_v[0]
    u01 = u_v[1]
    u10 = u_v[2]
    u11 = u_v[3]

    base = wid * _BLEND_W
    coff = wid * _COPY_W

    def off0(k):
        return 2 * _Q + base + k * _CH

    def off1(k):
        return 3 * _Q + base + k * _CH

    def offc(k):
        return coff + k * _CCH

    h_in = {}
    h_out = {}
    h_cin = {}
    h_cout = {}

    def start_in(k):
        b = k & 1
        h_in[k] = (
            pltpu.async_copy(state_hbm.at[pl.ds(off0(k), _CH)],
                             a0_v.at[b], si[b]),
            pltpu.async_copy(state_hbm.at[pl.ds(off1(k), _CH)],
                             a1_v.at[b], si[b]),
        )

    def start_cin(k):
        b = k & 1
        h_cin[k] = pltpu.async_copy(state_hbm.at[pl.ds(offc(k), _CCH)],
                                    c_v.at[b], ci[b])

    start_cin(0)
    start_cin(1)
    start_in(0)
    start_in(1)

    for k in range(_NCHUNK):
        b = k & 1
        if k >= 2:
            for h in h_out[k - 2]:
                h.wait()
        for h in h_in[k]:
            h.wait()

        a0b = a0_v.at[b]
        a1b = a1_v.at[b]
        o0b = o0_v.at[b]
        o1b = o1_v.at[b]

        @plsc.parallel_loop(0, _CH // _LANES, unroll=_UNROLL)
        def body(i):
            sl = pl.ds(i * _LANES, _LANES)
            a0 = a0b[sl]
            a1 = a1b[sl]
            o0b[sl] = u00 * a0 + u01 * a1
            o1b[sl] = u10 * a0 + u11 * a1

        h_out[k] = (
            pltpu.async_copy(o0b, out_hbm.at[pl.ds(off0(k), _CH)], so[b]),
            pltpu.async_copy(o1b, out_hbm.at[pl.ds(off1(k), _CH)], so[b]),
        )
        if k + 2 < _NCHUNK:
            start_in(k + 2)

        if k < _NCCH:
            if k >= 2:
                h_cout[k - 2].wait()
            h_cin[k].wait()
            h_cout[k] = pltpu.async_copy(c_v.at[b],
                                         out_hbm.at[pl.ds(offc(k), _CCH)],
                                         co[b])
            if k + 2 < _NCCH:
                start_cin(k + 2)

    for k in (_NCHUNK - 2, _NCHUNK - 1):
        for h in h_out[k]:
            h.wait()
    for k in (_NCCH - 2, _NCCH - 1):
        h_cout[k].wait()


def kernel(state, U):
    u_rows = jnp.tile(U.astype(jnp.float32).reshape(4, 1), (1, _LANES))
    return _cgate(state, u_rows)


# speedup vs baseline: 346.5079x; 1.0722x over previous
import functools

import jax
import jax.numpy as jnp
from jax import lax
from jax.experimental import pallas as pl
from jax.experimental.pallas import tpu as pltpu
from jax.experimental.pallas import tpu_sc as plsc

_NQ = 23
_DIM = 2 ** _NQ
_Q = _DIM // 4
_HALF = _DIM // 2
_NC = 2
_NS = 16
_NW = _NC * _NS
_BLEND_W = _Q // _NW
_CH = 8192
_NCHUNK = _BLEND_W // _CH
_CPC = _HALF // _NC
_SCH = 262144
_NSCH = _CPC // _SCH
_LANES = 16
_UNROLL = 8

_mesh = plsc.VectorSubcoreMesh(core_axis_name="c", subcore_axis_name="s")


@functools.partial(
    pl.kernel,
    mesh=_mesh,
    out_type=jax.ShapeDtypeStruct((_DIM,), jnp.float32),
    scratch_types=[
        pltpu.VMEM((4, _LANES), jnp.float32),
        pltpu.VMEM((2, _CH), jnp.float32),
        pltpu.VMEM((2, _CH), jnp.float32),
        pltpu.VMEM((2, _CH), jnp.float32),
        pltpu.VMEM((2, _CH), jnp.float32),
        pltpu.VMEM_SHARED((2, _SCH), jnp.float32),
        pltpu.SemaphoreType.DMA,
        pltpu.SemaphoreType.DMA,
        pltpu.SemaphoreType.DMA,
        pltpu.SemaphoreType.DMA,
        pltpu.SemaphoreType.DMA,
        pltpu.SemaphoreType.DMA,
        pltpu.SemaphoreType.DMA,
        pltpu.SemaphoreType.DMA,
    ],
)
def _cgate(state_hbm, u_hbm, out_hbm, u_v, a0_v, a1_v, o0_v, o1_v, s_v,
           si0, si1, so0, so1, ci0, ci1, co0, co1):
    si = (si0, si1)
    so = (so0, so1)
    ci = (ci0, ci1)
    co = (co0, co1)
    cid = lax.axis_index("c")
    sid = lax.axis_index("s")
    wid = sid * _NC + cid

    pltpu.sync_copy(u_hbm, u_v)
    u00 = u_v[0]
    u01 = u_v[1]
    u10 = u_v[2]
    u11 = u_v[3]

    base = wid * _BLEND_W
    cbase = cid * _CPC

    def off0(k):
        return 2 * _Q + base + k * _CH

    def off1(k):
        return 3 * _Q + base + k * _CH

    def cp_in(j):
        b = j & 1
        return pltpu.make_async_copy(
            state_hbm.at[pl.ds(cbase + j * _SCH, _SCH)], s_v.at[b], ci[b])

    def cp_out(j):
        b = j & 1
        return pltpu.make_async_copy(
            s_v.at[b], out_hbm.at[pl.ds(cbase + j * _SCH, _SCH)], co[b])

    h_in = {}
    h_out = {}

    def start_in(k):
        b = k & 1
        h_in[k] = (
            pltpu.async_copy(state_hbm.at[pl.ds(off0(k), _CH)],
                             a0_v.at[b], si[b]),
            pltpu.async_copy(state_hbm.at[pl.ds(off1(k), _CH)],
                             a1_v.at[b], si[b]),
        )

    @pl.when(sid == 0)
    def _():
        cp_in(0).start()
        cp_in(1).start()

    start_in(0)
    start_in(1)

    for k in range(_NCHUNK):
        b = k & 1
        if k >= 2:
            for h in h_out[k - 2]:
                h.wait()
        for h in h_in[k]:
            h.wait()

        a0b = a0_v.at[b]
        a1b = a1_v.at[b]
        o0b = o0_v.at[b]
        o1b = o1_v.at[b]

        @plsc.parallel_loop(0, _CH // _LANES, unroll=_UNROLL)
        def body(i):
            sl = pl.ds(i * _LANES, _LANES)
            a0 = a0b[sl]
            a1 = a1b[sl]
            o0b[sl] = u00 * a0 + u01 * a1
            o1b[sl] = u10 * a0 + u11 * a1

        h_out[k] = (
            pltpu.async_copy(o0b, out_hbm.at[pl.ds(off0(k), _CH)], so[b]),
            pltpu.async_copy(o1b, out_hbm.at[pl.ds(off1(k), _CH)], so[b]),
        )
        if k + 2 < _NCHUNK:
            start_in(k + 2)

        if k < _NSCH:
            @pl.when(sid == 0)
            def _():
                if k >= 2:
                    cp_out(k - 2).wait()
                cp_in(k).wait()
                cp_out(k).start()
                if k + 2 < _NSCH:
                    cp_in(k + 2).start()

    for k in (_NCHUNK - 2, _NCHUNK - 1):
        for h in h_out[k]:
            h.wait()

    @pl.when(sid == 0)
    def _():
        cp_out(_NSCH - 2).wait()
        cp_out(_NSCH - 1).wait()


def kernel(state, U):
    u_rows = jnp.tile(U.astype(jnp.float32).reshape(4, 1), (1, _LANES))
    return _cgate(state, u_rows)
